# baseline (speedup 1.0000x reference)
import functools

import jax
import jax.numpy as jnp
from jax import lax
from jax.experimental import pallas as pl
from jax.experimental.pallas import tpu as pltpu
from jax.experimental.pallas import tpu_sc as plsc

N = 10000
F = 128
F2 = F // 2
C = 16
E = 320000
NPAD = 10240
NW = 32
B = 80
KH = 126
BA = 112
KA = 180
EPAD = NW * KH * B
ZB = NPAD // 16
RB = 256
NCORES = 2
NSUB = 16

_mesh = plsc.VectorSubcoreMesh(
    core_axis_name="c", subcore_axis_name="s", num_cores=NCORES, num_subcores=NSUB
)


@functools.partial(
    pl.kernel,
    out_type=jax.ShapeDtypeStruct((NCORES, NPAD), jnp.float32),
    mesh=_mesh,
    scratch_types=[
        pltpu.VMEM((KH, B), jnp.int32),
        pltpu.VMEM((B,), jnp.float32),
        pltpu.VMEM((ZB,), jnp.float32),
        pltpu.VMEM_SHARED((NPAD,), jnp.float32),
        pltpu.SemaphoreType.DMA,
    ],
)
def _sc_hist(dst3, ones_h, z1d_h, hist_o, dstv, obuf, zbuf, acc, sem):
    cid = lax.axis_index("c")
    sid = lax.axis_index("s")
    wid = cid * NSUB + sid
    pltpu.sync_copy(z1d_h, zbuf)
    pltpu.sync_copy(ones_h, obuf)
    pltpu.sync_copy(dst3.at[wid], dstv)
    pltpu.sync_copy(zbuf, acc.at[pl.ds(sid * ZB, ZB)])
    plsc.subcore_barrier()

    CHUNK = 21

    def outer(j, carry):
        for t in range(CHUNK):
            pltpu.async_copy(obuf, acc.at[dstv.at[j * CHUNK + t]], sem, add=True)
        for t in range(CHUNK):
            pltpu.make_async_copy(obuf, acc.at[dstv.at[j * CHUNK + t]], sem).wait()
        return carry

    lax.fori_loop(0, KH // CHUNK, outer, 0)
    plsc.subcore_barrier()
    pltpu.sync_copy(acc.at[pl.ds(sid * ZB, ZB)], zbuf)
    pltpu.sync_copy(zbuf, hist_o.at[cid, pl.ds(sid * ZB, ZB)])


def _tc_xw_body(x_ref, w1_ref, xw_ref):
    xw_ref[...] = jnp.dot(
        x_ref[...], w1_ref[...], preferred_element_type=jnp.float32
    )


_tc_xw = pl.pallas_call(
    _tc_xw_body,
    grid=(NPAD // RB,),
    in_specs=[
        pl.BlockSpec((RB, F), lambda i: (i, 0)),
        pl.BlockSpec((F, F), lambda i: (0, 0)),
    ],
    out_specs=pl.BlockSpec((RB, F), lambda i: (i, 0)),
    out_shape=jax.ShapeDtypeStruct((NPAD, F), jnp.float32),
)


def _tc_dinvg_body(hist_ref, xw_ref, dinv_ref, g_ref):
    i = pl.program_id(0)
    hb = hist_ref[...]
    deg = hb[0] + hb[1] + 1.0
    r = i * RB + lax.broadcasted_iota(jnp.int32, (RB, 1), 0)
    dinv = jnp.where(r < N, deg ** -0.5, 0.0)
    dinv_ref[...] = dinv
    gfull = dinv * xw_ref[...]
    g_ref[0] = gfull[:, :F2]
    g_ref[1] = gfull[:, F2:]


_tc_dinvg = pl.pallas_call(
    _tc_dinvg_body,
    grid=(NPAD // RB,),
    in_specs=[
        pl.BlockSpec((2, RB, 1), lambda i: (0, i, 0)),
        pl.BlockSpec((RB, F), lambda i: (i, 0)),
    ],
    out_specs=[
        pl.BlockSpec((RB, 1), lambda i: (i, 0)),
        pl.BlockSpec((2, RB, F2), lambda i: (0, i, 0)),
    ],
    out_shape=[
        jax.ShapeDtypeStruct((NPAD, 1), jnp.float32),
        jax.ShapeDtypeStruct((NCORES, NPAD, F2), jnp.float32),
    ],
)


@functools.partial(
    pl.kernel,
    out_type=(
        jax.ShapeDtypeStruct((NCORES, NPAD, F2), jnp.float32),
        jax.ShapeDtypeStruct((NW, NPAD), jnp.float32),
    ),
    mesh=_mesh,
    scratch_types=[
        pltpu.VMEM((KA, BA), jnp.int32),
        pltpu.VMEM((KA, BA), jnp.int32),
        pltpu.VMEM((BA, F2), jnp.float32),
        pltpu.VMEM((BA, F2), jnp.float32),
        pltpu.VMEM((BA, F2), jnp.float32),
        pltpu.VMEM((ZB // 8, F2), jnp.float32),
        pltpu.VMEM((NPAD,), jnp.float32),
        pltpu.VMEM((NPAD,), jnp.float32),
        pltpu.VMEM_SHARED((NPAD, F2), jnp.float32),
        pltpu.SemaphoreType.DMA,
        pltpu.SemaphoreType.DMA,
        pltpu.SemaphoreType.DMA,
    ],
    compiler_params=pltpu.CompilerParams(
        use_tc_tiling_on_sc=False, needs_layout_passes=False
    ),
)
def _sc_agg(g_h, dinv_h, src3, dst3, z2d_h, znpad_h, agg_o, s_o,
            srcv, dstv, rb0, rb1, rb2, zbuf, dinvv, sloc, accS,
            sg0, sg1, sg2):
    cid = lax.axis_index("c")
    sid = lax.axis_index("s")
    wid = cid * NSUB + sid
    r0 = sid * ZB
    ZC = ZB // 8
    rbs = (rb0, rb1, rb2)
    sgs = (sg0, sg1, sg2)

    pltpu.sync_copy(z2d_h, zbuf)
    pltpu.sync_copy(src3.at[sid], srcv)
    pltpu.sync_copy(dst3.at[sid], dstv)
    pltpu.async_copy(g_h.at[cid].at[srcv.at[0]], rb0, sg0)
    pltpu.async_copy(g_h.at[cid].at[srcv.at[1]], rb1, sg1)
    pltpu.async_copy(g_h.at[cid].at[srcv.at[2]], rb2, sg2)
    pltpu.sync_copy(dinv_h, dinvv)
    pltpu.sync_copy(znpad_h, sloc)
    for j in range(8):
        pltpu.sync_copy(zbuf, accS.at[pl.ds(r0 + j * ZC, ZC)])
    plsc.subcore_barrier()

    def body(i, carry):
        for j in range(3):
            b = 3 * i + j
            rb, sg = rbs[j], sgs[j]
            for k in range(BA // 16):
                dt = dstv[b, pl.ds(k * 16, 16)]
                st = srcv[b, pl.ds(k * 16, 16)]
                vals = plsc.load_gather(dinvv, [dt])
                plsc.addupdate_scatter(sloc, [st], vals)
            pltpu.make_async_copy(g_h.at[cid].at[srcv.at[b]], rb, sg).wait()
            pltpu.sync_copy(rb, accS.at[dstv.at[b]], add=True)

            @pl.when(b + 3 < KA)
            def _():
                pltpu.async_copy(g_h.at[cid].at[srcv.at[b + 3]], rb, sg)
        return carry

    lax.fori_loop(0, KA // 3, body, 0)
    plsc.subcore_barrier()

    pltpu.sync_copy(sloc, s_o.at[wid])
    for j in range(8):
        pltpu.sync_copy(accS.at[pl.ds(r0 + j * ZC, ZC)], zbuf)
        pltpu.sync_copy(zbuf, agg_o.at[cid, pl.ds(r0 + j * ZC, ZC)])


def _tc_fin_body(agg_ref, g_ref, dinv_ref, s_ref, b1_ref, w2_ref, b2_ref,
                 out_ref, acc_ref):
    i = pl.program_id(0)
    aggb = jnp.concatenate([agg_ref[0], agg_ref[1]], axis=1)
    gb = jnp.concatenate([g_ref[0], g_ref[1]], axis=1)
    dcol = dinv_ref[...]
    out1 = dcol * (aggb + gb) + b1_ref[...]
    r = jnp.maximum(out1, 0.0)
    sb = 0.5 * jnp.sum(s_ref[...], axis=1, keepdims=True)
    cvec = dcol * (sb + dcol)
    part = lax.dot_general(cvec, r, (((0,), (0,)), ((), ())),
                           preferred_element_type=jnp.float32)

    @pl.when(i == 0)
    def _():
        acc_ref[...] = jnp.zeros_like(acc_ref)

    acc_ref[...] += part
    out_ref[...] = (
        jnp.dot(acc_ref[...], w2_ref[...], preferred_element_type=jnp.float32)
        * (1.0 / N) + b2_ref[...]
    )


_tc_fin = pl.pallas_call(
    _tc_fin_body,
    grid=(NPAD // RB,),
    in_specs=[
        pl.BlockSpec((2, RB, F2), lambda i: (0, i, 0)),
        pl.BlockSpec((2, RB, F2), lambda i: (0, i, 0)),
        pl.BlockSpec((RB, 1), lambda i: (i, 0)),
        pl.BlockSpec((RB, NW), lambda i: (i, 0)),
        pl.BlockSpec((1, F), lambda i: (0, 0)),
        pl.BlockSpec((F, C), lambda i: (0, 0)),
        pl.BlockSpec((1, C), lambda i: (0, 0)),
    ],
    out_specs=pl.BlockSpec((1, C), lambda i: (0, 0)),
    out_shape=jax.ShapeDtypeStruct((1, C), jnp.float32),
    scratch_shapes=[pltpu.VMEM((1, F), jnp.float32)],
)


def kernel(x, edge_index, W1, b1, W2, b2):
    ei = edge_index.astype(jnp.int32)
    pad = jnp.full((EPAD - E,), N, jnp.int32)
    srcf = jnp.concatenate([ei[0], pad])
    dstf = jnp.concatenate([ei[1], pad])
    xp = jnp.pad(x, ((0, NPAD - N), (0, 0)))
    ones_b = jnp.ones((B,), jnp.float32)
    z1 = jnp.zeros((ZB,), jnp.float32)
    z2 = jnp.zeros((ZB // 8, F2), jnp.float32)

    znpad = jnp.zeros((NPAD,), jnp.float32)

    xw = _tc_xw(xp, W1)
    hist = _sc_hist(dstf.reshape(NW, KH, B), ones_b, z1)
    dinv2d, g = _tc_dinvg(hist.reshape(NCORES, NPAD, 1), xw)
    agg, sfull = _sc_agg(g, dinv2d.reshape(NPAD),
                         srcf.reshape(NSUB, KA, BA),
                         dstf.reshape(NSUB, KA, BA), z2, znpad)
    out = _tc_fin(agg, g, dinv2d, sfull.T,
                  b1.reshape(1, F), W2, b2.reshape(1, C))
    return out


# speedup vs baseline: 1.0907x; 1.0907x over previous
import functools

import jax
import jax.numpy as jnp
from jax import lax
from jax.experimental import pallas as pl
from jax.experimental.pallas import tpu as pltpu
from jax.experimental.pallas import tpu_sc as plsc

N = 10000
F = 128
F2 = F // 2
C = 16
E = 320000
NPAD = 10240
NW = 32
B = 80
KH = 126
BA = 112
KA = 180
EPAD = NW * KH * B
ZB = NPAD // 16
RB = 256
NCORES = 2
NSUB = 16

_mesh = plsc.VectorSubcoreMesh(
    core_axis_name="c", subcore_axis_name="s", num_cores=NCORES, num_subcores=NSUB
)


@functools.partial(
    pl.kernel,
    out_type=jax.ShapeDtypeStruct((NCORES, NPAD), jnp.float32),
    mesh=_mesh,
    scratch_types=[
        pltpu.VMEM((KH, B), jnp.int32),
        pltpu.VMEM((B,), jnp.float32),
        pltpu.VMEM((ZB,), jnp.float32),
        pltpu.VMEM_SHARED((NPAD,), jnp.float32),
        pltpu.SemaphoreType.DMA,
    ],
)
def _sc_hist(dst3, ones_h, z1d_h, hist_o, dstv, obuf, zbuf, acc, sem):
    cid = lax.axis_index("c")
    sid = lax.axis_index("s")
    wid = cid * NSUB + sid
    pltpu.sync_copy(z1d_h, zbuf)
    pltpu.sync_copy(ones_h, obuf)
    pltpu.sync_copy(dst3.at[wid], dstv)
    pltpu.sync_copy(zbuf, acc.at[pl.ds(sid * ZB, ZB)])
    plsc.subcore_barrier()

    CHUNK = 21

    def outer(j, carry):
        for t in range(CHUNK):
            pltpu.async_copy(obuf, acc.at[dstv.at[j * CHUNK + t]], sem, add=True)
        for t in range(CHUNK):
            pltpu.make_async_copy(obuf, acc.at[dstv.at[j * CHUNK + t]], sem).wait()
        return carry

    lax.fori_loop(0, KH // CHUNK, outer, 0)
    plsc.subcore_barrier()
    pltpu.sync_copy(acc.at[pl.ds(sid * ZB, ZB)], zbuf)
    pltpu.sync_copy(zbuf, hist_o.at[cid, pl.ds(sid * ZB, ZB)])


def _tc_dinvg_body(hist_ref, x_ref, w1_ref, dinv_ref, g_ref):
    i = pl.program_id(0)
    hb = hist_ref[...]
    deg = hb[0] + hb[1] + 1.0
    r = i * RB + lax.broadcasted_iota(jnp.int32, (RB, 1), 0)
    dinv = jnp.where(r < N, deg ** -0.5, 0.0)
    dinv_ref[...] = dinv
    gfull = dinv * jnp.dot(
        x_ref[...], w1_ref[...], preferred_element_type=jnp.float32
    )
    g_ref[0] = gfull[:, :F2]
    g_ref[1] = gfull[:, F2:]


_tc_dinvg = pl.pallas_call(
    _tc_dinvg_body,
    grid=(NPAD // RB,),
    in_specs=[
        pl.BlockSpec((2, RB, 1), lambda i: (0, i, 0)),
        pl.BlockSpec((RB, F), lambda i: (i, 0)),
        pl.BlockSpec((F, F), lambda i: (0, 0)),
    ],
    out_specs=[
        pl.BlockSpec((RB, 1), lambda i: (i, 0)),
        pl.BlockSpec((2, RB, F2), lambda i: (0, i, 0)),
    ],
    out_shape=[
        jax.ShapeDtypeStruct((NPAD, 1), jnp.float32),
        jax.ShapeDtypeStruct((NCORES, NPAD, F2), jnp.float32),
    ],
)


@functools.partial(
    pl.kernel,
    out_type=(
        jax.ShapeDtypeStruct((NCORES, NPAD, F2), jnp.float32),
        jax.ShapeDtypeStruct((NW, NPAD), jnp.float32),
    ),
    mesh=_mesh,
    scratch_types=[
        pltpu.VMEM((KA, BA), jnp.int32),
        pltpu.VMEM((KA, BA), jnp.int32),
        pltpu.VMEM((BA, F2), jnp.float32),
        pltpu.VMEM((BA, F2), jnp.float32),
        pltpu.VMEM((BA, F2), jnp.float32),
        pltpu.VMEM((ZB // 8, F2), jnp.float32),
        pltpu.VMEM((NPAD,), jnp.float32),
        pltpu.VMEM((NPAD,), jnp.float32),
        pltpu.VMEM_SHARED((NPAD, F2), jnp.float32),
        pltpu.SemaphoreType.DMA,
        pltpu.SemaphoreType.DMA,
        pltpu.SemaphoreType.DMA,
    ],
    compiler_params=pltpu.CompilerParams(
        use_tc_tiling_on_sc=False, needs_layout_passes=False
    ),
)
def _sc_agg(g_h, dinv_h, src3, dst3, z2d_h, znpad_h, agg_o, s_o,
            srcv, dstv, rb0, rb1, rb2, zbuf, dinvv, sloc, accS,
            sg0, sg1, sg2):
    cid = lax.axis_index("c")
    sid = lax.axis_index("s")
    wid = cid * NSUB + sid
    r0 = sid * ZB
    ZC = ZB // 8
    rbs = (rb0, rb1, rb2)
    sgs = (sg0, sg1, sg2)

    pltpu.sync_copy(z2d_h, zbuf)
    pltpu.sync_copy(src3.at[sid], srcv)
    pltpu.sync_copy(dst3.at[sid], dstv)
    pltpu.async_copy(g_h.at[cid].at[srcv.at[0]], rb0, sg0)
    pltpu.async_copy(g_h.at[cid].at[srcv.at[1]], rb1, sg1)
    pltpu.async_copy(g_h.at[cid].at[srcv.at[2]], rb2, sg2)
    pltpu.sync_copy(dinv_h, dinvv)
    pltpu.sync_copy(znpad_h, sloc)
    for j in range(8):
        pltpu.sync_copy(zbuf, accS.at[pl.ds(r0 + j * ZC, ZC)])
    plsc.subcore_barrier()

    def body(i, carry):
        for j in range(3):
            b = 3 * i + j
            rb, sg = rbs[j], sgs[j]
            for k in range(BA // 16):
                dt = dstv[b, pl.ds(k * 16, 16)]
                st = srcv[b, pl.ds(k * 16, 16)]
                vals = plsc.load_gather(dinvv, [dt])
                plsc.addupdate_scatter(sloc, [st], vals)
            pltpu.make_async_copy(g_h.at[cid].at[srcv.at[b]], rb, sg).wait()
            pltpu.sync_copy(rb, accS.at[dstv.at[b]], add=True)

            @pl.when(b + 3 < KA)
            def _():
                pltpu.async_copy(g_h.at[cid].at[srcv.at[b + 3]], rb, sg)
        return carry

    lax.fori_loop(0, KA // 3, body, 0)
    plsc.subcore_barrier()

    pltpu.sync_copy(sloc, s_o.at[wid])
    for j in range(8):
        pltpu.sync_copy(accS.at[pl.ds(r0 + j * ZC, ZC)], zbuf)
        pltpu.sync_copy(zbuf, agg_o.at[cid, pl.ds(r0 + j * ZC, ZC)])


def _tc_fin_body(agg_ref, g_ref, dinv_ref, s_ref, b1_ref, w2_ref, b2_ref,
                 out_ref, acc_ref):
    i = pl.program_id(0)
    aggb = jnp.concatenate([agg_ref[0], agg_ref[1]], axis=1)
    gb = jnp.concatenate([g_ref[0], g_ref[1]], axis=1)
    dcol = dinv_ref[...]
    out1 = dcol * (aggb + gb) + b1_ref[...]
    r = jnp.maximum(out1, 0.0)
    sb = 0.5 * jnp.sum(s_ref[...], axis=1, keepdims=True)
    cvec = dcol * (sb + dcol)
    part = lax.dot_general(cvec, r, (((0,), (0,)), ((), ())),
                           preferred_element_type=jnp.float32)

    @pl.when(i == 0)
    def _():
        acc_ref[...] = jnp.zeros_like(acc_ref)

    acc_ref[...] += part
    out_ref[...] = (
        jnp.dot(acc_ref[...], w2_ref[...], preferred_element_type=jnp.float32)
        * (1.0 / N) + b2_ref[...]
    )


_tc_fin = pl.pallas_call(
    _tc_fin_body,
    grid=(NPAD // RB,),
    in_specs=[
        pl.BlockSpec((2, RB, F2), lambda i: (0, i, 0)),
        pl.BlockSpec((2, RB, F2), lambda i: (0, i, 0)),
        pl.BlockSpec((RB, 1), lambda i: (i, 0)),
        pl.BlockSpec((RB, NW), lambda i: (i, 0)),
        pl.BlockSpec((1, F), lambda i: (0, 0)),
        pl.BlockSpec((F, C), lambda i: (0, 0)),
        pl.BlockSpec((1, C), lambda i: (0, 0)),
    ],
    out_specs=pl.BlockSpec((1, C), lambda i: (0, 0)),
    out_shape=jax.ShapeDtypeStruct((1, C), jnp.float32),
    scratch_shapes=[pltpu.VMEM((1, F), jnp.float32)],
)


def kernel(x, edge_index, W1, b1, W2, b2):
    ei = edge_index.astype(jnp.int32)
    pad = jnp.full((EPAD - E,), N, jnp.int32)
    srcf = jnp.concatenate([ei[0], pad])
    dstf = jnp.concatenate([ei[1], pad])
    xp = jnp.pad(x, ((0, NPAD - N), (0, 0)))
    ones_b = jnp.ones((B,), jnp.float32)
    z1 = jnp.zeros((ZB,), jnp.float32)
    z2 = jnp.zeros((ZB // 8, F2), jnp.float32)

    znpad = jnp.zeros((NPAD,), jnp.float32)

    hist = _sc_hist(dstf.reshape(NW, KH, B), ones_b, z1)
    dinv2d, g = _tc_dinvg(hist.reshape(NCORES, NPAD, 1), xp, W1)
    agg, sfull = _sc_agg(g, dinv2d.reshape(NPAD),
                         srcf.reshape(NSUB, KA, BA),
                         dstf.reshape(NSUB, KA, BA), z2, znpad)
    out = _tc_fin(agg, g, dinv2d, sfull.T,
                  b1.reshape(1, F), W2, b2.reshape(1, C))
    return out


# speedup vs baseline: 1.1063x; 1.0143x over previous
import functools

import jax
import jax.numpy as jnp
from jax import lax
from jax.experimental import pallas as pl
from jax.experimental.pallas import tpu as pltpu
from jax.experimental.pallas import tpu_sc as plsc

N = 10000
F = 128
F2 = F // 2
C = 16
E = 320000
NPAD = 10240
NW = 32
B = 80
KH = 126
BA = 112
KA = 180
EPAD = NW * KH * B
ZB = NPAD // 16
RB = 256
NCORES = 2
NSUB = 16

_mesh = plsc.VectorSubcoreMesh(
    core_axis_name="c", subcore_axis_name="s", num_cores=NCORES, num_subcores=NSUB
)


@functools.partial(
    pl.kernel,
    out_type=jax.ShapeDtypeStruct((NCORES, NPAD), jnp.float32),
    mesh=_mesh,
    scratch_types=[
        pltpu.VMEM((KH, B), jnp.int32),
        pltpu.VMEM((B,), jnp.float32),
        pltpu.VMEM((ZB,), jnp.float32),
        pltpu.VMEM_SHARED((NPAD,), jnp.float32),
        pltpu.SemaphoreType.DMA,
    ],
)
def _sc_hist(dst3, ones_h, z1d_h, hist_o, dstv, obuf, zbuf, acc, sem):
    cid = lax.axis_index("c")
    sid = lax.axis_index("s")
    wid = cid * NSUB + sid
    pltpu.sync_copy(z1d_h, zbuf)
    pltpu.sync_copy(ones_h, obuf)
    pltpu.sync_copy(dst3.at[wid], dstv)
    pltpu.sync_copy(zbuf, acc.at[pl.ds(sid * ZB, ZB)])
    plsc.subcore_barrier()

    CHUNK = 21

    def outer(j, carry):
        for t in range(CHUNK):
            pltpu.async_copy(obuf, acc.at[dstv.at[j * CHUNK + t]], sem, add=True)
        for t in range(CHUNK):
            pltpu.make_async_copy(obuf, acc.at[dstv.at[j * CHUNK + t]], sem).wait()
        return carry

    lax.fori_loop(0, KH // CHUNK, outer, 0)
    plsc.subcore_barrier()
    pltpu.sync_copy(acc.at[pl.ds(sid * ZB, ZB)], zbuf)
    pltpu.sync_copy(zbuf, hist_o.at[cid, pl.ds(sid * ZB, ZB)])


def _tc_dinvg_body(hist_ref, x_ref, w1_ref, dinv_ref, g_ref):
    i = pl.program_id(0)
    hb = hist_ref[...]
    deg = hb[0] + hb[1] + 1.0
    r = i * RB + lax.broadcasted_iota(jnp.int32, (RB, 1), 0)
    dinv = jnp.where(r < N, deg ** -0.5, 0.0)
    dinv_ref[...] = dinv
    gfull = dinv * jnp.dot(
        x_ref[...], w1_ref[...], preferred_element_type=jnp.float32
    )
    g_ref[0] = gfull[:, :F2]
    g_ref[1] = gfull[:, F2:]


_tc_dinvg = pl.pallas_call(
    _tc_dinvg_body,
    grid=(NPAD // RB,),
    in_specs=[
        pl.BlockSpec((2, RB, 1), lambda i: (0, i, 0)),
        pl.BlockSpec((RB, F), lambda i: (i, 0)),
        pl.BlockSpec((F, F), lambda i: (0, 0)),
    ],
    out_specs=[
        pl.BlockSpec((RB, 1), lambda i: (i, 0)),
        pl.BlockSpec((2, RB, F2), lambda i: (0, i, 0)),
    ],
    out_shape=[
        jax.ShapeDtypeStruct((NPAD, 1), jnp.float32),
        jax.ShapeDtypeStruct((NCORES, NPAD, F2), jnp.float32),
    ],
)


@functools.partial(
    pl.kernel,
    out_type=(
        jax.ShapeDtypeStruct((NCORES, NPAD, F2), jnp.float32),
        jax.ShapeDtypeStruct((NW, NPAD), jnp.float32),
    ),
    mesh=_mesh,
    scratch_types=[
        pltpu.VMEM((KA, BA), jnp.int32),
        pltpu.VMEM((KA, BA), jnp.int32),
        pltpu.VMEM((BA, F2), jnp.float32),
        pltpu.VMEM((BA, F2), jnp.float32),
        pltpu.VMEM((BA, F2), jnp.float32),
        pltpu.VMEM((ZB // 8, F2), jnp.float32),
        pltpu.VMEM((NPAD,), jnp.float32),
        pltpu.VMEM((NPAD,), jnp.float32),
        pltpu.VMEM_SHARED((NPAD, F2), jnp.float32),
        pltpu.SemaphoreType.DMA,
        pltpu.SemaphoreType.DMA,
        pltpu.SemaphoreType.DMA,
    ],
    compiler_params=pltpu.CompilerParams(
        use_tc_tiling_on_sc=False, needs_layout_passes=False
    ),
)
def _sc_agg(g_h, dinv_h, src3, dst3, z2d_h, znpad_h, agg_o, s_o,
            srcv, dstv, rb0, rb1, rb2, zbuf, dinvv, sloc, accS,
            sg0, sg1, sg2):
    cid = lax.axis_index("c")
    sid = lax.axis_index("s")
    wid = cid * NSUB + sid
    r0 = sid * ZB
    ZC = ZB // 8
    rbs = (rb0, rb1, rb2)
    sgs = (sg0, sg1, sg2)

    pltpu.sync_copy(z2d_h, zbuf)
    pltpu.sync_copy(src3.at[sid], srcv)
    pltpu.sync_copy(dst3.at[sid], dstv)
    pltpu.async_copy(g_h.at[cid].at[srcv.at[0]], rb0, sg0)
    pltpu.async_copy(g_h.at[cid].at[srcv.at[1]], rb1, sg1)
    pltpu.async_copy(g_h.at[cid].at[srcv.at[2]], rb2, sg2)
    pltpu.sync_copy(dinv_h, dinvv)
    pltpu.sync_copy(znpad_h, sloc)
    for j in range(8):
        pltpu.sync_copy(zbuf, accS.at[pl.ds(r0 + j * ZC, ZC)])
    plsc.subcore_barrier()

    def body(i, carry):
        for j in range(3):
            b = 3 * i + j
            rb, sg = rbs[j], sgs[j]
            for k in range(BA // 16):
                dt = dstv[b, pl.ds(k * 16, 16)]
                st = srcv[b, pl.ds(k * 16, 16)]
                vals = plsc.load_gather(dinvv, [dt])
                plsc.addupdate_scatter(sloc, [st], vals)
            pltpu.make_async_copy(g_h.at[cid].at[srcv.at[b]], rb, sg).wait()
            pltpu.sync_copy(rb, accS.at[dstv.at[b]], add=True)

            @pl.when(b + 3 < KA)
            def _():
                pltpu.async_copy(g_h.at[cid].at[srcv.at[b + 3]], rb, sg)
        return carry

    lax.fori_loop(0, KA // 3, body, 0)
    plsc.subcore_barrier()

    pltpu.sync_copy(sloc, s_o.at[wid])
    for j in range(8):
        pltpu.sync_copy(accS.at[pl.ds(r0 + j * ZC, ZC)], zbuf)
        pltpu.sync_copy(zbuf, agg_o.at[cid, pl.ds(r0 + j * ZC, ZC)])


def _tc_fin_body(agg_ref, g_ref, dinv_ref, s_ref, dinvt_ref,
                 b1_ref, w2_ref, b2_ref, out_ref, acc_ref):
    i = pl.program_id(0)
    aggb = jnp.concatenate([agg_ref[0], agg_ref[1]], axis=1)
    gb = jnp.concatenate([g_ref[0], g_ref[1]], axis=1)
    dcol = dinv_ref[...]
    out1 = dcol * (aggb + gb) + b1_ref[...]
    r = jnp.maximum(out1, 0.0)
    drow = dinvt_ref[...]
    srow = 0.5 * jnp.sum(s_ref[...], axis=0, keepdims=True)
    crow = drow * (srow + drow)
    part = jnp.dot(crow, r, preferred_element_type=jnp.float32)

    @pl.when(i == 0)
    def _():
        acc_ref[...] = jnp.zeros_like(acc_ref)

    acc_ref[...] += part
    out_ref[...] = (
        jnp.dot(acc_ref[...], w2_ref[...], preferred_element_type=jnp.float32)
        * (1.0 / N) + b2_ref[...]
    )


_tc_fin = pl.pallas_call(
    _tc_fin_body,
    grid=(NPAD // RB,),
    in_specs=[
        pl.BlockSpec((2, RB, F2), lambda i: (0, i, 0)),
        pl.BlockSpec((2, RB, F2), lambda i: (0, i, 0)),
        pl.BlockSpec((RB, 1), lambda i: (i, 0)),
        pl.BlockSpec((NW, RB), lambda i: (0, i)),
        pl.BlockSpec((1, RB), lambda i: (0, i)),
        pl.BlockSpec((1, F), lambda i: (0, 0)),
        pl.BlockSpec((F, C), lambda i: (0, 0)),
        pl.BlockSpec((1, C), lambda i: (0, 0)),
    ],
    out_specs=pl.BlockSpec((1, C), lambda i: (0, 0)),
    out_shape=jax.ShapeDtypeStruct((1, C), jnp.float32),
    scratch_shapes=[pltpu.VMEM((1, F), jnp.float32)],
)


def kernel(x, edge_index, W1, b1, W2, b2):
    ei = edge_index.astype(jnp.int32)
    pad = jnp.full((EPAD - E,), N, jnp.int32)
    srcf = jnp.concatenate([ei[0], pad])
    dstf = jnp.concatenate([ei[1], pad])
    xp = jnp.pad(x, ((0, NPAD - N), (0, 0)))
    ones_b = jnp.ones((B,), jnp.float32)
    z1 = jnp.zeros((ZB,), jnp.float32)
    z2 = jnp.zeros((ZB // 8, F2), jnp.float32)

    znpad = jnp.zeros((NPAD,), jnp.float32)

    hist = _sc_hist(dstf.reshape(NW, KH, B), ones_b, z1)
    dinv2d, g = _tc_dinvg(hist.reshape(NCORES, NPAD, 1), xp, W1)
    agg, sfull = _sc_agg(g, dinv2d.reshape(NPAD),
                         srcf.reshape(NSUB, KA, BA),
                         dstf.reshape(NSUB, KA, BA), z2, znpad)
    out = _tc_fin(agg, g, dinv2d, sfull, dinv2d.reshape(1, NPAD),
                  b1.reshape(1, F), W2, b2.reshape(1, C))
    return out


# speedup vs baseline: 1.1149x; 1.0078x over previous
import functools

import jax
import jax.numpy as jnp
from jax import lax
from jax.experimental import pallas as pl
from jax.experimental.pallas import tpu as pltpu
from jax.experimental.pallas import tpu_sc as plsc

N = 10000
F = 128
F2 = F // 2
C = 16
E = 320000
NPAD = 10240
NW = 32
B = 80
KH = 126
BA = 112
KA = 180
EPAD = NW * KH * B
ZB = NPAD // 16
RB = 256
NCORES = 2
NSUB = 16

_mesh = plsc.VectorSubcoreMesh(
    core_axis_name="c", subcore_axis_name="s", num_cores=NCORES, num_subcores=NSUB
)


@functools.partial(
    pl.kernel,
    out_type=jax.ShapeDtypeStruct((NCORES, NPAD), jnp.float32),
    mesh=_mesh,
    scratch_types=[
        pltpu.VMEM((KH, B), jnp.int32),
        pltpu.VMEM((B,), jnp.float32),
        pltpu.VMEM((ZB,), jnp.float32),
        pltpu.VMEM_SHARED((NPAD,), jnp.float32),
        pltpu.SemaphoreType.DMA,
    ],
)
def _sc_hist(dst3, ones_h, z1d_h, hist_o, dstv, obuf, zbuf, acc, sem):
    cid = lax.axis_index("c")
    sid = lax.axis_index("s")
    wid = cid * NSUB + sid
    pltpu.sync_copy(z1d_h, zbuf)
    pltpu.sync_copy(ones_h, obuf)
    pltpu.sync_copy(dst3.at[wid], dstv)
    pltpu.sync_copy(zbuf, acc.at[pl.ds(sid * ZB, ZB)])
    plsc.subcore_barrier()

    CHUNK = 21

    def outer(j, carry):
        for t in range(CHUNK):
            pltpu.async_copy(obuf, acc.at[dstv.at[j * CHUNK + t]], sem, add=True)
        for t in range(CHUNK):
            pltpu.make_async_copy(obuf, acc.at[dstv.at[j * CHUNK + t]], sem).wait()
        return carry

    lax.fori_loop(0, KH // CHUNK, outer, 0)
    plsc.subcore_barrier()
    pltpu.sync_copy(acc.at[pl.ds(sid * ZB, ZB)], zbuf)
    pltpu.sync_copy(zbuf, hist_o.at[cid, pl.ds(sid * ZB, ZB)])


def _tc_dinvg_body(hist_ref, x_ref, w1_ref, dinv_ref, g_ref):
    i = pl.program_id(0)
    hb = hist_ref[...]
    deg = hb[0] + hb[1] + 1.0
    r = i * RB + lax.broadcasted_iota(jnp.int32, (RB, 1), 0)
    dinv = jnp.where(r < N, deg ** -0.5, 0.0)
    dinv_ref[...] = dinv
    gfull = dinv * jnp.dot(
        x_ref[...], w1_ref[...], preferred_element_type=jnp.float32
    )
    g_ref[0] = gfull[:, :F2]
    g_ref[1] = gfull[:, F2:]


_tc_dinvg = pl.pallas_call(
    _tc_dinvg_body,
    grid=(NPAD // RB,),
    in_specs=[
        pl.BlockSpec((2, RB, 1), lambda i: (0, i, 0)),
        pl.BlockSpec((RB, F), lambda i: (i, 0)),
        pl.BlockSpec((F, F), lambda i: (0, 0)),
    ],
    out_specs=[
        pl.BlockSpec((RB, 1), lambda i: (i, 0)),
        pl.BlockSpec((2, RB, F2), lambda i: (0, i, 0)),
    ],
    out_shape=[
        jax.ShapeDtypeStruct((NPAD, 1), jnp.float32),
        jax.ShapeDtypeStruct((NCORES, NPAD, F2), jnp.float32),
    ],
)


@functools.partial(
    pl.kernel,
    out_type=(
        jax.ShapeDtypeStruct((NCORES, NPAD, F2), jnp.float32),
        jax.ShapeDtypeStruct((NW, NPAD), jnp.float32),
    ),
    mesh=_mesh,
    scratch_types=[
        pltpu.VMEM((KA, BA), jnp.int32),
        pltpu.VMEM((KA, BA), jnp.int32),
        pltpu.VMEM((BA, F2), jnp.float32),
        pltpu.VMEM((BA, F2), jnp.float32),
        pltpu.VMEM((BA, F2), jnp.float32),
        pltpu.VMEM((ZB // 8, F2), jnp.float32),
        pltpu.VMEM((NPAD,), jnp.float32),
        pltpu.VMEM((NPAD,), jnp.float32),
        pltpu.VMEM_SHARED((NPAD, F2), jnp.float32),
        pltpu.SemaphoreType.DMA,
        pltpu.SemaphoreType.DMA,
        pltpu.SemaphoreType.DMA,
    ],
    compiler_params=pltpu.CompilerParams(
        use_tc_tiling_on_sc=False, needs_layout_passes=False
    ),
)
def _sc_agg(g_h, dinv_h, src3, dst3, z2d_h, znpad_h, agg_o, s_o,
            srcv, dstv, rb0, rb1, rb2, zbuf, dinvv, sloc, accS,
            sg0, sg1, sg2):
    cid = lax.axis_index("c")
    sid = lax.axis_index("s")
    wid = cid * NSUB + sid
    r0 = sid * ZB
    ZC = ZB // 8
    rbs = (rb0, rb1, rb2)
    sgs = (sg0, sg1, sg2)

    pltpu.sync_copy(z2d_h, zbuf)
    pltpu.sync_copy(src3.at[sid], srcv)
    pltpu.sync_copy(dst3.at[sid], dstv)
    pltpu.async_copy(g_h.at[cid].at[srcv.at[0]], rb0, sg0)
    pltpu.async_copy(g_h.at[cid].at[srcv.at[1]], rb1, sg1)
    pltpu.async_copy(g_h.at[cid].at[srcv.at[2]], rb2, sg2)
    pltpu.sync_copy(dinv_h, dinvv)
    pltpu.sync_copy(znpad_h, sloc)
    for j in range(8):
        pltpu.sync_copy(zbuf, accS.at[pl.ds(r0 + j * ZC, ZC)])
    plsc.subcore_barrier()

    def body(i, carry):
        for j in range(3):
            b = 3 * i + j
            rb, sg = rbs[j], sgs[j]
            for k in range(BA // 16):
                dt = dstv[b, pl.ds(k * 16, 16)]
                st = srcv[b, pl.ds(k * 16, 16)]
                vals = plsc.load_gather(dinvv, [dt])
                plsc.addupdate_scatter(sloc, [st], vals)
            pltpu.make_async_copy(g_h.at[cid].at[srcv.at[b]], rb, sg).wait()
            pltpu.sync_copy(rb, accS.at[dstv.at[b]], add=True)

            @pl.when(b + 3 < KA)
            def _():
                pltpu.async_copy(g_h.at[cid].at[srcv.at[b + 3]], rb, sg)
        return carry

    lax.fori_loop(0, KA // 3, body, 0)
    plsc.subcore_barrier()

    pltpu.sync_copy(sloc, s_o.at[wid])
    pltpu.sync_copy(accS.at[pl.ds(r0, ZB)], agg_o.at[cid, pl.ds(r0, ZB)])


def _tc_fin_body(agg_ref, g_ref, dinv_ref, s_ref, dinvt_ref,
                 b1_ref, w2_ref, b2_ref, out_ref, acc_ref):
    i = pl.program_id(0)
    aggb = jnp.concatenate([agg_ref[0], agg_ref[1]], axis=1)
    gb = jnp.concatenate([g_ref[0], g_ref[1]], axis=1)
    dcol = dinv_ref[...]
    out1 = dcol * (aggb + gb) + b1_ref[...]
    r = jnp.maximum(out1, 0.0)
    drow = dinvt_ref[...]
    srow = 0.5 * jnp.sum(s_ref[...], axis=0, keepdims=True)
    crow = drow * (srow + drow)
    part = jnp.dot(crow, r, preferred_element_type=jnp.float32)

    @pl.when(i == 0)
    def _():
        acc_ref[...] = jnp.zeros_like(acc_ref)

    acc_ref[...] += part
    out_ref[...] = (
        jnp.dot(acc_ref[...], w2_ref[...], preferred_element_type=jnp.float32)
        * (1.0 / N) + b2_ref[...]
    )


_tc_fin = pl.pallas_call(
    _tc_fin_body,
    grid=(NPAD // RB,),
    in_specs=[
        pl.BlockSpec((2, RB, F2), lambda i: (0, i, 0)),
        pl.BlockSpec((2, RB, F2), lambda i: (0, i, 0)),
        pl.BlockSpec((RB, 1), lambda i: (i, 0)),
        pl.BlockSpec((NW, RB), lambda i: (0, i)),
        pl.BlockSpec((1, RB), lambda i: (0, i)),
        pl.BlockSpec((1, F), lambda i: (0, 0)),
        pl.BlockSpec((F, C), lambda i: (0, 0)),
        pl.BlockSpec((1, C), lambda i: (0, 0)),
    ],
    out_specs=pl.BlockSpec((1, C), lambda i: (0, 0)),
    out_shape=jax.ShapeDtypeStruct((1, C), jnp.float32),
    scratch_shapes=[pltpu.VMEM((1, F), jnp.float32)],
)


def kernel(x, edge_index, W1, b1, W2, b2):
    ei = edge_index.astype(jnp.int32)
    pad = jnp.full((EPAD - E,), N, jnp.int32)
    srcf = jnp.concatenate([ei[0], pad])
    dstf = jnp.concatenate([ei[1], pad])
    xp = jnp.pad(x, ((0, NPAD - N), (0, 0)))
    ones_b = jnp.ones((B,), jnp.float32)
    z1 = jnp.zeros((ZB,), jnp.float32)
    z2 = jnp.zeros((ZB // 8, F2), jnp.float32)

    znpad = jnp.zeros((NPAD,), jnp.float32)

    hist = _sc_hist(dstf.reshape(NW, KH, B), ones_b, z1)
    dinv2d, g = _tc_dinvg(hist.reshape(NCORES, NPAD, 1), xp, W1)
    agg, sfull = _sc_agg(g, dinv2d.reshape(NPAD),
                         srcf.reshape(NSUB, KA, BA),
                         dstf.reshape(NSUB, KA, BA), z2, znpad)
    out = _tc_fin(agg, g, dinv2d, sfull, dinv2d.reshape(1, NPAD),
                  b1.reshape(1, F), W2, b2.reshape(1, C))
    return out
